# baseline (speedup 1.0000x reference)
import jax
import jax.numpy as jnp
from jax.experimental import pallas as pl
from jax.experimental.pallas import tpu as pltpu


def _gru_body(T, BB, nb,
              xt_ref, wh_ref, wm_ref, wzu_ref, wru_ref, wiu_ref,
              wzh_ref, wrh_ref, wih_ref, bh_ref, bz_ref, br_ref, bi_ref,
              out_ref, h_scr, gz_scr, gr_scr, gi_scr):
    H = wh_ref.shape[1]
    j = pl.program_id(1)
    prev = (j - 1) % 2
    cur = j % 2

    h = h_scr[...]
    for t in range(T):
        hb = h.astype(jnp.bfloat16)
        z = jax.nn.sigmoid(
            gz_scr[prev, t * BB:(t + 1) * BB, :]
            + jnp.dot(hb, wzh_ref[...], preferred_element_type=jnp.float32)
        )
        r = jax.nn.sigmoid(
            gr_scr[prev, t * BB:(t + 1) * BB, :]
            + jnp.dot(hb, wrh_ref[...], preferred_element_type=jnp.float32)
        )
        hp = jnp.tanh(
            gi_scr[prev, t * BB:(t + 1) * BB, :]
            + jnp.dot((r * h).astype(jnp.bfloat16), wih_ref[...],
                      preferred_element_type=jnp.float32)
        )
        h = (1.0 - z) * h + z * hp
    h_scr[...] = h

    @pl.when(j == 0)
    def _init():
        h_scr[...] = (
            jnp.dot(xt_ref[0], wh_ref[...], preferred_element_type=jnp.float32)
            + bh_ref[...]
        )

    xb = xt_ref[...].reshape(T * BB, xt_ref.shape[2])
    u = jnp.dot(xb, wm_ref[...], preferred_element_type=jnp.float32)
    u_bf = u.astype(jnp.bfloat16)
    gz_scr[cur] = (
        jnp.dot(u_bf, wzu_ref[...], preferred_element_type=jnp.float32)
        + bz_ref[...]
    )
    gr_scr[cur] = (
        jnp.dot(u_bf, wru_ref[...], preferred_element_type=jnp.float32)
        + br_ref[...]
    )
    gi_scr[cur] = (
        jnp.dot(u_bf, wiu_ref[...], preferred_element_type=jnp.float32)
        + bi_ref[...]
    )

    @pl.when(j == nb)
    def _fin():
        out_ref[...] = h


def kernel(x, Wm, bm, Wh, bh, Wz, bz, Wr, br, Wi, bi):
    B, S, D = x.shape
    H = Wm.shape[0]
    T = 16
    NB = S // T
    NCORES = 2
    BB = B // NCORES

    bf = jnp.bfloat16
    xt = jnp.swapaxes(x, 0, 1).astype(bf)
    wh = Wh.T.astype(bf)
    wm = Wm.T.astype(bf)
    wzu = Wz[:, :H].T.astype(bf)
    wru = Wr[:, :H].T.astype(bf)
    wiu = Wi[:, :H].T.astype(bf)
    wzh = Wz[:, H:].T.astype(bf)
    wrh = Wr[:, H:].T.astype(bf)
    wih = Wi[:, H:].T.astype(bf)
    bz_eff = (bz + bm @ Wz[:, :H].T).reshape(1, H)
    br_eff = (br + bm @ Wr[:, :H].T).reshape(1, H)
    bi_eff = (bi + bm @ Wi[:, :H].T).reshape(1, H)
    bh2 = bh.reshape(1, H)

    full = lambda a: pl.BlockSpec(a.shape, lambda i, j: (0,) * a.ndim)
    xmap = lambda i, j: (jnp.minimum(j, NB - 1), i, 0)

    out = pl.pallas_call(
        lambda *refs: _gru_body(T, BB, NB, *refs),
        grid=(NCORES, NB + 1),
        in_specs=[
            pl.BlockSpec((T, BB, D), xmap),
            full(wh), full(wm),
            full(wzu), full(wru), full(wiu),
            full(wzh), full(wrh), full(wih),
            full(bh2), full(bz_eff), full(br_eff), full(bi_eff),
        ],
        out_specs=pl.BlockSpec((BB, H), lambda i, j: (i, 0)),
        out_shape=jax.ShapeDtypeStruct((B, H), jnp.float32),
        scratch_shapes=[
            pltpu.VMEM((BB, H), jnp.float32),
            pltpu.VMEM((2, T * BB, H), jnp.float32),
            pltpu.VMEM((2, T * BB, H), jnp.float32),
            pltpu.VMEM((2, T * BB, H), jnp.float32),
        ],
        compiler_params=pltpu.CompilerParams(
            dimension_semantics=("parallel", "arbitrary"),
            vmem_limit_bytes=60 * 1024 * 1024,
        ),
    )(xt, wh, wm, wzu, wru, wiu, wzh, wrh, wih, bh2, bz_eff, br_eff, bi_eff)

    return out[:, None, :]


# speedup vs baseline: 1.6019x; 1.6019x over previous
import jax
import jax.numpy as jnp
from jax.experimental import pallas as pl
from jax.experimental.pallas import tpu as pltpu


def _gru_body(T, B, nb,
              xt_ref, wh_ref, wm_ref, wu_ref, wzr_ref, wih_ref,
              bh_ref, bg_ref,
              out_ref, h_scr, g_scr):
    H = wh_ref.shape[1]
    j = pl.program_id(0)

    @pl.when(j == 0)
    def _init():
        h_scr[...] = (
            jnp.dot(xt_ref[0], wh_ref[...], preferred_element_type=jnp.float32)
            + bh_ref[...]
        )

    xb = xt_ref[...].reshape(T * B, xt_ref.shape[2])
    u = jnp.dot(xb, wm_ref[...], preferred_element_type=jnp.float32)
    u_bf = u.astype(jnp.bfloat16)
    g_scr[...] = (
        jnp.dot(u_bf, wu_ref[...], preferred_element_type=jnp.float32)
        + bg_ref[...]
    ).astype(jnp.bfloat16)

    h = h_scr[...]
    for t in range(T):
        hb = h.astype(jnp.bfloat16)
        rows = g_scr[t * B:(t + 1) * B, :]
        zr = jax.nn.sigmoid(
            rows[:, :2 * H]
            + jnp.dot(hb, wzr_ref[...], preferred_element_type=jnp.float32)
        )
        z = zr[:, :H]
        r = zr[:, H:]
        hp = jnp.tanh(
            rows[:, 2 * H:]
            + jnp.dot((r * h).astype(jnp.bfloat16), wih_ref[...],
                      preferred_element_type=jnp.float32)
        )
        h = (1.0 - z) * h + z * hp
    h_scr[...] = h

    @pl.when(j == nb - 1)
    def _fin():
        out_ref[...] = h


def kernel(x, Wm, bm, Wh, bh, Wz, bz, Wr, br, Wi, bi):
    B, S, D = x.shape
    H = Wm.shape[0]
    T = 16
    NB = S // T

    bf = jnp.bfloat16
    xt = jnp.swapaxes(x, 0, 1).astype(bf)
    wh = Wh.T.astype(bf)
    wm = Wm.T.astype(bf)
    wu = jnp.concatenate(
        [Wz[:, :H].T, Wr[:, :H].T, Wi[:, :H].T], axis=1).astype(bf)
    wzr = jnp.concatenate([Wz[:, H:].T, Wr[:, H:].T], axis=1).astype(bf)
    wih = Wi[:, H:].T.astype(bf)
    bg = jnp.concatenate([
        (bz + bm @ Wz[:, :H].T),
        (br + bm @ Wr[:, :H].T),
        (bi + bm @ Wi[:, :H].T),
    ]).reshape(1, 3 * H)
    bh2 = bh.reshape(1, H)

    full = lambda a: pl.BlockSpec(a.shape, lambda j: (0,) * a.ndim)

    out = pl.pallas_call(
        lambda *refs: _gru_body(T, B, NB, *refs),
        grid=(NB,),
        in_specs=[
            pl.BlockSpec((T, B, D), lambda j: (j, 0, 0)),
            full(wh), full(wm), full(wu), full(wzr), full(wih),
            full(bh2), full(bg),
        ],
        out_specs=pl.BlockSpec((B, H), lambda j: (0, 0)),
        out_shape=jax.ShapeDtypeStruct((B, H), jnp.float32),
        scratch_shapes=[
            pltpu.VMEM((B, H), jnp.float32),
            pltpu.VMEM((T * B, 3 * H), jnp.bfloat16),
        ],
        compiler_params=pltpu.CompilerParams(
            dimension_semantics=("arbitrary",),
            vmem_limit_bytes=60 * 1024 * 1024,
        ),
    )(xt, wh, wm, wu, wzr, wih, bh2, bg)

    return out[:, None, :]


# speedup vs baseline: 1.6102x; 1.0051x over previous
import jax
import jax.numpy as jnp
from jax.experimental import pallas as pl
from jax.experimental.pallas import tpu as pltpu


def _gru_body(T, B, nb,
              xt_ref, wmh_ref, wg_ref, bh_ref, bg_ref,
              out_ref, h_scr, g_scr):
    H = wg_ref.shape[0] // 2
    j = pl.program_id(0)

    @pl.when(j == 0)
    def _init():
        h_scr[...] = (
            jnp.dot(xt_ref[0], wmh_ref[:, H:],
                    preferred_element_type=jnp.float32)
            + bh_ref[...]
        )

    xb = xt_ref[...].reshape(T * B, xt_ref.shape[2])
    u = jnp.dot(xb, wmh_ref[:, :H], preferred_element_type=jnp.float32)
    u_bf = u.astype(jnp.bfloat16)
    g_scr[...] = (
        jnp.dot(u_bf, wg_ref[:H, :], preferred_element_type=jnp.float32)
        + bg_ref[...]
    ).astype(jnp.bfloat16)

    h = h_scr[...]
    for t in range(T):
        hb = h.astype(jnp.bfloat16)
        rows = g_scr[t * B:(t + 1) * B, :]
        zr = jax.nn.sigmoid(
            rows[:, :2 * H]
            + jnp.dot(hb, wg_ref[H:, :2 * H],
                      preferred_element_type=jnp.float32)
        )
        z = zr[:, :H]
        r = zr[:, H:]
        hp = jnp.tanh(
            rows[:, 2 * H:]
            + jnp.dot((r * h).astype(jnp.bfloat16), wg_ref[H:, 2 * H:],
                      preferred_element_type=jnp.float32)
        )
        h = (1.0 - z) * h + z * hp
    h_scr[...] = h

    @pl.when(j == nb - 1)
    def _fin():
        out_ref[...] = h


def kernel(x, Wm, bm, Wh, bh, Wz, bz, Wr, br, Wi, bi):
    B, S, D = x.shape
    H = Wm.shape[0]
    T = 16
    NB = S // T

    bf = jnp.bfloat16
    xt = jnp.swapaxes(x, 0, 1).astype(bf)
    wmh = jnp.concatenate([Wm, Wh], axis=0).astype(bf).T
    wg = jnp.concatenate([Wz, Wr, Wi], axis=0).astype(bf).T
    bg = jnp.concatenate([
        (bz + bm @ Wz[:, :H].T),
        (br + bm @ Wr[:, :H].T),
        (bi + bm @ Wi[:, :H].T),
    ]).reshape(1, 3 * H)
    bh2 = bh.reshape(1, H)

    full = lambda a: pl.BlockSpec(a.shape, lambda j: (0,) * a.ndim)

    out = pl.pallas_call(
        lambda *refs: _gru_body(T, B, NB, *refs),
        grid=(NB,),
        in_specs=[
            pl.BlockSpec((T, B, D), lambda j: (j, 0, 0)),
            full(wmh), full(wg), full(bh2), full(bg),
        ],
        out_specs=pl.BlockSpec((B, H), lambda j: (0, 0)),
        out_shape=jax.ShapeDtypeStruct((B, H), jnp.float32),
        scratch_shapes=[
            pltpu.VMEM((B, H), jnp.float32),
            pltpu.VMEM((T * B, 3 * H), jnp.bfloat16),
        ],
        compiler_params=pltpu.CompilerParams(
            dimension_semantics=("arbitrary",),
            vmem_limit_bytes=60 * 1024 * 1024,
        ),
    )(xt, wmh, wg, bh2, bg)

    return out[:, None, :]


# speedup vs baseline: 1.6113x; 1.0007x over previous
import jax
import jax.numpy as jnp
from jax.experimental import pallas as pl
from jax.experimental.pallas import tpu as pltpu


def _gru_body(T, B, nb,
              xt_ref, wmh_hbm, wg_hbm, bh_ref, bg_ref,
              out_ref, h_scr, g_scr, wmh_ref, wg_ref, sem):
    H = wg_ref.shape[0] // 2
    j = pl.program_id(0)

    @pl.when(j == 0)
    def _load_weights():
        cp0 = pltpu.make_async_copy(wmh_hbm, wmh_ref, sem.at[0])
        cp1 = pltpu.make_async_copy(wg_hbm, wg_ref, sem.at[1])
        cp0.start()
        cp1.start()
        cp0.wait()
        cp1.wait()
        h_scr[...] = (
            jnp.dot(xt_ref[0], wmh_ref[:, H:],
                    preferred_element_type=jnp.float32)
            + bh_ref[...]
        )

    xb = xt_ref[...].reshape(T * B, xt_ref.shape[2])
    u = jnp.dot(xb, wmh_ref[:, :H], preferred_element_type=jnp.float32)
    u_bf = u.astype(jnp.bfloat16)
    g_scr[...] = (
        jnp.dot(u_bf, wg_ref[:H, :], preferred_element_type=jnp.float32)
        + bg_ref[...]
    ).astype(jnp.bfloat16)

    h = h_scr[...]
    for t in range(T):
        hb = h.astype(jnp.bfloat16)
        rows = g_scr[t * B:(t + 1) * B, :]
        zr = jax.nn.sigmoid(
            rows[:, :2 * H]
            + jnp.dot(hb, wg_ref[H:, :2 * H],
                      preferred_element_type=jnp.float32)
        )
        z = zr[:, :H]
        r = zr[:, H:]
        hp = jnp.tanh(
            rows[:, 2 * H:]
            + jnp.dot((r * h).astype(jnp.bfloat16), wg_ref[H:, 2 * H:],
                      preferred_element_type=jnp.float32)
        )
        h = (1.0 - z) * h + z * hp
    h_scr[...] = h

    @pl.when(j == nb - 1)
    def _fin():
        out_ref[...] = h


def kernel(x, Wm, bm, Wh, bh, Wz, bz, Wr, br, Wi, bi):
    B, S, D = x.shape
    H = Wm.shape[0]
    T = 16
    NB = S // T

    bf = jnp.bfloat16
    xt = jnp.swapaxes(x, 0, 1).astype(bf)
    wmh = jnp.concatenate([Wm, Wh], axis=0).astype(bf).T
    wg = jnp.concatenate([Wz, Wr, Wi], axis=0).astype(bf).T
    bg = jnp.concatenate([
        (bz + bm @ Wz[:, :H].T),
        (br + bm @ Wr[:, :H].T),
        (bi + bm @ Wi[:, :H].T),
    ]).reshape(1, 3 * H)
    bh2 = bh.reshape(1, H)

    full = lambda a: pl.BlockSpec(a.shape, lambda j: (0,) * a.ndim)

    out = pl.pallas_call(
        lambda *refs: _gru_body(T, B, NB, *refs),
        grid=(NB,),
        in_specs=[
            pl.BlockSpec((T, B, D), lambda j: (j, 0, 0)),
            pl.BlockSpec(memory_space=pl.ANY),
            pl.BlockSpec(memory_space=pl.ANY),
            full(bh2), full(bg),
        ],
        out_specs=pl.BlockSpec((B, H), lambda j: (0, 0)),
        out_shape=jax.ShapeDtypeStruct((B, H), jnp.float32),
        scratch_shapes=[
            pltpu.VMEM((B, H), jnp.float32),
            pltpu.VMEM((T * B, 3 * H), jnp.bfloat16),
            pltpu.VMEM((D, 2 * H), jnp.bfloat16),
            pltpu.VMEM((2 * H, 3 * H), jnp.bfloat16),
            pltpu.SemaphoreType.DMA((2,)),
        ],
        compiler_params=pltpu.CompilerParams(
            dimension_semantics=("arbitrary",),
            vmem_limit_bytes=60 * 1024 * 1024,
        ),
    )(xt, wmh, wg, bh2, bg)

    return out[:, None, :]


# speedup vs baseline: 1.7380x; 1.0786x over previous
import jax
import jax.numpy as jnp
from jax.experimental import pallas as pl
from jax.experimental.pallas import tpu as pltpu


def _gru_body(T, B, nb,
              x_ref, wmh_hbm, wg_hbm, bh_ref, bg_ref,
              out_ref, h_scr, g_scr, wmh_ref, wg_ref, sem):
    H = wg_ref.shape[0] // 2
    j = pl.program_id(0)

    @pl.when(j == 0)
    def _load_weights():
        cp0 = pltpu.make_async_copy(wmh_hbm, wmh_ref, sem.at[0])
        cp1 = pltpu.make_async_copy(wg_hbm, wg_ref, sem.at[1])
        cp0.start()
        cp1.start()
        cp0.wait()
        cp1.wait()
        h_scr[...] = (
            jnp.dot(x_ref[:, 0, :].astype(jnp.bfloat16), wmh_ref[:, H:],
                    preferred_element_type=jnp.float32)
            + bh_ref[...]
        )

    xb = jnp.swapaxes(x_ref[...], 0, 1).astype(jnp.bfloat16)
    xb = xb.reshape(T * B, x_ref.shape[2])
    u = jnp.dot(xb, wmh_ref[:, :H], preferred_element_type=jnp.float32)
    u_bf = u.astype(jnp.bfloat16)
    g_scr[...] = (
        jnp.dot(u_bf, wg_ref[:H, :], preferred_element_type=jnp.float32)
        + bg_ref[...]
    ).astype(jnp.bfloat16)

    h = h_scr[...]
    for t in range(T):
        hb = h.astype(jnp.bfloat16)
        rows = g_scr[t * B:(t + 1) * B, :]
        zr = jax.nn.sigmoid(
            rows[:, :2 * H]
            + jnp.dot(hb, wg_ref[H:, :2 * H],
                      preferred_element_type=jnp.float32)
        )
        z = zr[:, :H]
        r = zr[:, H:]
        hp = jnp.tanh(
            rows[:, 2 * H:]
            + jnp.dot((r * h).astype(jnp.bfloat16), wg_ref[H:, 2 * H:],
                      preferred_element_type=jnp.float32)
        )
        h = (1.0 - z) * h + z * hp
    h_scr[...] = h

    @pl.when(j == nb - 1)
    def _fin():
        out_ref[...] = h


def kernel(x, Wm, bm, Wh, bh, Wz, bz, Wr, br, Wi, bi):
    B, S, D = x.shape
    H = Wm.shape[0]
    T = 16
    NB = S // T

    bf = jnp.bfloat16
    wmh = jnp.concatenate([Wm, Wh], axis=0).astype(bf).T
    wg = jnp.concatenate([Wz, Wr, Wi], axis=0).astype(bf).T
    bg = jnp.concatenate([
        (bz + bm @ Wz[:, :H].T),
        (br + bm @ Wr[:, :H].T),
        (bi + bm @ Wi[:, :H].T),
    ]).reshape(1, 3 * H)
    bh2 = bh.reshape(1, H)

    full = lambda a: pl.BlockSpec(a.shape, lambda j: (0,) * a.ndim)

    out = pl.pallas_call(
        lambda *refs: _gru_body(T, B, NB, *refs),
        grid=(NB,),
        in_specs=[
            pl.BlockSpec((B, T, D), lambda j: (0, j, 0)),
            pl.BlockSpec(memory_space=pl.ANY),
            pl.BlockSpec(memory_space=pl.ANY),
            full(bh2), full(bg),
        ],
        out_specs=pl.BlockSpec((B, H), lambda j: (0, 0)),
        out_shape=jax.ShapeDtypeStruct((B, H), jnp.float32),
        scratch_shapes=[
            pltpu.VMEM((B, H), jnp.float32),
            pltpu.VMEM((T * B, 3 * H), jnp.bfloat16),
            pltpu.VMEM((D, 2 * H), jnp.bfloat16),
            pltpu.VMEM((2 * H, 3 * H), jnp.bfloat16),
            pltpu.SemaphoreType.DMA((2,)),
        ],
        compiler_params=pltpu.CompilerParams(
            dimension_semantics=("arbitrary",),
            vmem_limit_bytes=60 * 1024 * 1024,
        ),
    )(x, wmh, wg, bh2, bg)

    return out[:, None, :]
